# baseline (speedup 1.0000x reference)
import functools

import jax
import jax.numpy as jnp
from jax import lax
from jax.experimental import pallas as pl
from jax.experimental.pallas import tpu as pltpu
from jax.experimental.pallas import tpu_sc as plsc

F32 = jnp.float32
I32 = jnp.int32
EPS = 1e-5

NC, NS, L = 2, 16, 16
NW = NC * NS
D = 304
NKEY = 10000
NKP = 10016
SEG = NKP // NW
NPTS = 50000
E = 160000
EW = 5120
EP = NW * EW
BCH = 128
SCH = 1280
GSUB = 64
NV = D // L
R = 1280
NSTEP = E // R

_mesh = plsc.VectorSubcoreMesh(core_axis_name="c", subcore_axis_name="s",
                               num_cores=NC, num_subcores=NS)


def _wid():
    return lax.axis_index("s") * NC + lax.axis_index("c")


@functools.partial(
    pl.kernel,
    out_type=jax.ShapeDtypeStruct((EP, 16), F32),
    mesh=_mesh,
    compiler_params=pltpu.CompilerParams(needs_layout_passes=False,
                                         use_tc_tiling_on_sc=False),
    scratch_types=[
        pltpu.VMEM((NKP,), I32),
        pltpu.VMEM((BCH,), I32),
        pltpu.VMEM((BCH,), I32),
        pltpu.VMEM((BCH,), I32),
        pltpu.VMEM((BCH, 16), F32),
        pltpu.VMEM((BCH, 16), F32),
        pltpu.SemaphoreType.DMA,
        pltpu.SemaphoreType.DMA,
    ],
)
def _sc_build_a(pt_hbm, pc16_hbm, k1_hbm, src_hbm, dst_hbm, out_hbm,
                k1v, srcb, dstb, pkib, ra, rb, sem1, sem2):
    w = _wid()
    base = w * EW
    pltpu.sync_copy(k1_hbm, k1v)

    def chunk(c, _):
        off = base + c * BCH
        pltpu.sync_copy(src_hbm.at[pl.ds(off, BCH)], srcb)
        pltpu.sync_copy(dst_hbm.at[pl.ds(off, BCH)], dstb)

        def g16(i, _):
            idx = dstb[pl.ds(i * L, L)]
            pkib[pl.ds(i * L, L)] = plsc.load_gather(k1v, [idx])
            return 0

        lax.fori_loop(0, BCH // L, g16, 0)
        cpa = pltpu.async_copy(pt_hbm.at[srcb], ra, sem1)
        cpb = pltpu.async_copy(pc16_hbm.at[pkib], rb, sem2)
        cpa.wait()
        cpb.wait()

        def rowfix(r, _):
            ra[r, :] = ra[r, :] - rb[r, :]
            return 0

        lax.fori_loop(0, BCH, rowfix, 0)
        pltpu.sync_copy(ra, out_hbm.at[pl.ds(off, BCH)])
        return 0

    lax.fori_loop(0, EW // BCH, chunk, 0)


@functools.partial(
    pl.kernel,
    out_type=jax.ShapeDtypeStruct((EP, D), F32),
    mesh=_mesh,
    compiler_params=pltpu.CompilerParams(needs_layout_passes=False,
                                         use_tc_tiling_on_sc=False),
    scratch_types=[
        pltpu.VMEM((BCH,), I32),
        pltpu.VMEM((BCH,), I32),
        pltpu.VMEM((BCH, D), F32),
        pltpu.VMEM((BCH, 16), F32),
        pltpu.SemaphoreType.DMA,
        pltpu.SemaphoreType.DMA,
    ],
)
def _sc_build_g(t_hbm, ctr_hbm, src_hbm, dst_hbm, out_hbm,
                srcb, dstb, ra, rb, sem1, sem2):
    w = _wid()
    base = w * EW

    def chunk(c, _):
        off = base + c * BCH
        pltpu.sync_copy(src_hbm.at[pl.ds(off, BCH)], srcb)
        pltpu.sync_copy(dst_hbm.at[pl.ds(off, BCH)], dstb)
        cpa = pltpu.async_copy(t_hbm.at[srcb], ra, sem1)
        cpb = pltpu.async_copy(ctr_hbm.at[dstb], rb, sem2)
        cpa.wait()
        cpb.wait()

        def rowfix(r, _):
            ra[r, pl.ds(D - 16, 16)] = ra[r, pl.ds(D - 16, 16)] - rb[r, :]
            return 0

        lax.fori_loop(0, BCH, rowfix, 0)
        pltpu.sync_copy(ra, out_hbm.at[pl.ds(off, BCH)])
        return 0

    lax.fori_loop(0, EW // BCH, chunk, 0)


@functools.partial(
    pl.kernel,
    out_type=jax.ShapeDtypeStruct((NKP * D,), F32),
    mesh=_mesh,
    compiler_params=pltpu.CompilerParams(needs_layout_passes=False,
                                         use_tc_tiling_on_sc=False),
    scratch_types=[
        pltpu.VMEM((SEG * D,), F32),
        pltpu.VMEM((GSUB, D), F32),
        pltpu.VMEM((SCH,), I32),
        pltpu.VMEM((SCH // GSUB, GSUB), I32),
        pltpu.VMEM((SCH + L,), I32),
        pltpu.VMEM((8, D), F32),
        pltpu.SemaphoreType.DMA,
    ],
)
def _sc_scatter_max(y_hbm, dst_hbm, stats_hbm, out_hbm,
                    acc, rows, dstb, idb, ldb, statv, sem):
    w = _wid()
    lo = w * SEG
    pltpu.sync_copy(stats_hbm, statv)

    def initr(i, _):
        b = i * D
        for v in range(NV):
            acc[pl.ds(b + v * L, L)] = statv[4, pl.ds(v * L, L)]
        return 0

    lax.fori_loop(0, SEG, initr, 0)

    zid = jnp.zeros((L,), I32)

    def initz(j, _):
        for v in range(GSUB // L):
            idb[j, pl.ds(v * L, L)] = zid
        return 0

    lax.fori_loop(0, SCH // GSUB, initz, 0)

    iot = lax.iota(I32, L)

    def chunk(k, _):
        pltpu.sync_copy(dst_hbm.at[pl.ds(k * SCH, SCH)], dstb)

        def scan(i, cnt):
            d = dstb[pl.ds(i * L, L)]
            rel = d - lo
            m = (rel >= 0) & (rel < SEG)
            pos = cnt + plsc.cumsum(m.astype(I32)) - 1
            e = (k * SCH + i * L) + iot
            plsc.store_scatter(idb, [pos >> 6, pos & 63], e, mask=m)
            plsc.store_scatter(ldb, [pos], rel, mask=m)
            return cnt + plsc.all_reduce_population_count(m)

        cntv = lax.fori_loop(0, SCH // L, scan, jnp.zeros((L,), I32))
        cnt = jnp.max(cntv)
        nsub = (cnt + (GSUB - 1)) // GSUB

        def sub(j, _):
            pltpu.async_copy(y_hbm.at[idb.at[j]], rows, sem).wait()
            nr = jnp.minimum(cnt - j * GSUB, GSUB)

            def row(r, _):
                ld = ldb[pl.ds(j * GSUB + r, L)][0]
                ab = ld * D
                for v in range(NV):
                    sa = pl.ds(ab + v * L, L)
                    acc[sa] = jnp.maximum(acc[sa], rows[r, pl.ds(v * L, L)])
                return 0

            lax.fori_loop(0, nr, row, 0)
            return 0

        lax.fori_loop(0, nsub, sub, 0)
        return 0

    lax.fori_loop(0, EP // SCH, chunk, 0)

    def flushr(i, _):
        b = i * D
        for v in range(NV):
            sq = statv[0, pl.ds(v * L, L)]
            g = statv[1, pl.ds(v * L, L)]
            be = statv[2, pl.ds(v * L, L)]
            mean = statv[3, pl.ds(v * L, L)]
            sl = pl.ds(b + v * L, L)
            acc[sl] = (acc[sl] - mean) / sq * g + be
        return 0

    lax.fori_loop(0, SEG, flushr, 0)
    pltpu.sync_copy(acc, out_hbm.at[pl.ds(lo * D, SEG * D)])


def _edge_layer(X, stats_in, Wp, brow, pvec):
    K = X.shape[1]
    N = Wp.shape[1]

    def body(x_ref, si_ref, w_ref, b_ref, p_ref, y_ref, so_ref, accs):
        i = pl.program_id(0)
        x = (x_ref[...] - si_ref[3:4, :]) / si_ref[0:1, :] * si_ref[1:2, :] \
            + si_ref[2:3, :]
        y = jnp.dot(x, w_ref[...], preferred_element_type=F32) + b_ref[...]
        y = jnp.maximum(y, 0.0)
        y_ref[...] = y
        ps = jnp.concatenate([jnp.sum(y, 0, keepdims=True),
                              jnp.sum(y * y, 0, keepdims=True)], 0)
        prev = jnp.where(i == 0, jnp.zeros_like(ps), accs[0:2, :])
        tot = prev + ps
        accs[0:2, :] = tot

        @pl.when(i == NSTEP - 1)
        def _():
            mean = tot[0:1, :] / float(E)
            var = tot[1:2, :] / float(E) - mean * mean
            sq = jnp.sqrt(var + EPS)
            g = p_ref[0:1, :]
            be = p_ref[1:2, :]
            gs = jnp.where(g != 0.0, g, 1.0)
            init = jnp.where(g != 0.0, mean - be * sq / gs, mean)
            so_ref[...] = jnp.concatenate(
                [sq, g, be, mean, init, jnp.zeros((3, N), F32)], 0)

    return pl.pallas_call(
        body,
        grid=(NSTEP,),
        in_specs=[pl.BlockSpec((R, K), lambda i: (i, 0)),
                  pl.BlockSpec((8, K), lambda i: (0, 0)),
                  pl.BlockSpec((K, N), lambda i: (0, 0)),
                  pl.BlockSpec((1, N), lambda i: (0, 0)),
                  pl.BlockSpec((8, N), lambda i: (0, 0))],
        out_specs=[pl.BlockSpec((R, N), lambda i: (i, 0)),
                   pl.BlockSpec((8, N), lambda i: (0, 0))],
        out_shape=[jax.ShapeDtypeStruct((E, N), F32),
                   jax.ShapeDtypeStruct((8, N), F32)],
        scratch_shapes=[pltpu.VMEM((8, N), F32)],
        compiler_params=pltpu.CompilerParams(
            dimension_semantics=("arbitrary",)),
    )(X, stats_in, Wp, brow, pvec)


def _kp_block(X, specs, add=None, final=None):
    ins = [X]
    for Wp, pv in specs:
        ins += [Wp, pv]
    if add is not None:
        ins.append(add)
    if final is not None:
        ins += [final[0], final[1]]
    ns = len(specs)
    na = add is not None
    nf = final is not None

    def body(*refs):
        it = iter(refs)
        x = next(it)[...]
        sp = [(next(it), next(it)) for _ in range(ns)]
        a = next(it) if na else None
        f = (next(it), next(it)) if nf else None
        o = next(it)
        rmask = lax.broadcasted_iota(I32, (x.shape[0], 1), 0) < NKEY
        h = x
        for w_ref, p_ref in sp:
            hh = jnp.dot(h, w_ref[...], preferred_element_type=F32)
            hh = jnp.maximum(hh + p_ref[0:1, :], 0.0)
            hm = jnp.where(rmask, hh, 0.0)
            mean = jnp.sum(hm, 0, keepdims=True) / float(NKEY)
            dev = jnp.where(rmask, hh - mean, 0.0)
            var = jnp.sum(dev * dev, 0, keepdims=True) / float(NKEY)
            h = (hh - mean) / jnp.sqrt(var + EPS) * p_ref[1:2, :] \
                + p_ref[2:3, :]
        if a is not None:
            h = h + a[...]
        if f is not None:
            h = jnp.dot(h, f[0][...], preferred_element_type=F32) + f[1][...]
        o[...] = h

    if nf:
        nout = final[0].shape[1]
    elif na:
        nout = add.shape[1]
    else:
        nout = specs[-1][0].shape[1]
    return pl.pallas_call(
        body,
        out_shape=jax.ShapeDtypeStruct((NKP, nout), F32),
        compiler_params=pltpu.CompilerParams(
            vmem_limit_bytes=100 * 1024 * 1024),
    )(*ins)


def _pw(W, K, N, cs=0):
    return jnp.zeros((K, N), F32).at[:W.shape[0], cs:cs + W.shape[1]].set(W)


def _pr(b, N, cs=0):
    return jnp.zeros((1, N), F32).at[0, cs:cs + b.shape[0]].set(b)


def _pp(b, g, be, N, cs=0):
    n0 = b.shape[0]
    out = jnp.zeros((8, N), F32)
    out = out.at[0, cs:cs + n0].set(b)
    out = out.at[1, cs:cs + n0].set(g)
    out = out.at[2, cs:cs + n0].set(be)
    return out


def _pgb(g, be, N):
    n0 = g.shape[0]
    out = jnp.zeros((8, N), F32)
    out = out.at[0, :n0].set(g)
    out = out.at[1, :n0].set(be)
    return out


def _id_stats(K):
    return jnp.zeros((8, K), F32).at[0, :].set(1.0).at[1, :].set(1.0)


def kernel(point_features, point_coordinates, keypoint_indices_1, set_indices,
           keypoint_coordinates, keypoint_indices_2, edges, params):
    pf = point_features.astype(F32)
    pc = point_coordinates.astype(F32)
    kc = keypoint_coordinates.astype(F32)
    k1 = keypoint_indices_1[:, 0].astype(I32)
    si = set_indices.astype(I32)
    eg = edges.astype(I32)

    z12 = jnp.zeros((NPTS, 12), F32)
    PT = jnp.concatenate([pf, pc, z12], 1)
    PC16 = jnp.concatenate([jnp.zeros((NPTS, 1), F32), pc, z12], 1)
    k1p = jnp.concatenate([k1, jnp.zeros((NKP - NKEY,), I32)])
    padE = jnp.zeros((EP - E,), I32)
    srcA = jnp.concatenate([si[:, 0], padE])
    dstA = jnp.concatenate([si[:, 1], padE])
    dstAn = jnp.concatenate([si[:, 1], padE - 1])
    srcG = jnp.concatenate([eg[:, 0], padE])
    dstG = jnp.concatenate([eg[:, 1], padE])
    dstGn = jnp.concatenate([eg[:, 1], padE - 1])

    X0 = _sc_build_a(PT, PC16, k1p, srcA, dstA)
    dims1 = [16, 32, 64, 128, D]
    Y = X0
    st = _id_stats(16)
    for li, (W, b, g, be) in enumerate(params["point_linears"]):
        Y, st = _edge_layer(Y, st, _pw(W, dims1[li], dims1[li + 1]),
                            _pr(b, dims1[li + 1]), _pgb(g, be, dims1[li + 1]))
    agg = _sc_scatter_max(Y, dstAn, st).reshape(NKP, D)

    (W1, b1, g1, be1), (W2, b2, g2, be2) = params["out_linears"]
    coordsP = jnp.zeros((NKP, D), F32).at[:NKEY, 300:303].set(kc)
    h = _kp_block(agg, [(_pw(W1, D, D), _pp(b1, g1, be1, D))])
    feats = _kp_block(h, [(_pw(W2, D, D), _pp(b2, g2, be2, D))], add=coordsP)

    coords16 = jnp.zeros((NKP, 16), F32).at[:NKEY, 12:15].set(kc)
    for gp in params["graph_nets"]:
        (aW1, ab1, ag1, abe1), (aW2, ab2, ag2, abe2) = gp["auto_offset"]
        h1 = _kp_block(feats, [(_pw(aW1, D, 64), _pp(ab1, ag1, abe1, 64))])
        ctrT = _kp_block(h1, [(_pw(aW2, 64, 16, cs=12),
                               _pp(ab2, ag2, abe2, 16, cs=12))], add=coords16)
        Xe = _sc_build_g(feats, ctrT, srcG, dstG)
        (eW1, eb1, eg1, ebe1), (eW2, eb2, eg2, ebe2) = gp["edge"]
        Y1, st1 = _edge_layer(Xe, _id_stats(D), _pw(eW1, D, D),
                              _pr(eb1, D), _pgb(eg1, ebe1, D))
        Y2, st2 = _edge_layer(Y1, st1, _pw(eW2, D, D),
                              _pr(eb2, D), _pgb(eg2, ebe2, D))
        agge = _sc_scatter_max(Y2, dstGn, st2).reshape(NKP, D)
        (uW1, ub1, ug1, ube1), (uW2, ub2, ug2, ube2) = gp["update"]
        hu = _kp_block(agge, [(_pw(uW1, D, D), _pp(ub1, ug1, ube1, D))])
        feats = _kp_block(hu, [(_pw(uW2, D, D), _pp(ub2, ug2, ube2, D))],
                          add=feats)

    (cW, cb, cg, cbe), = params["cls_mlp"]
    cfW, cfb = params["cls_final"]
    logits_p = _kp_block(feats, [(_pw(cW, D, 64), _pp(cb, cg, cbe, 64))],
                         final=(_pw(cfW, 64, 8), _pr(cfb, 8)))
    boxes = []
    for lc in params["loc"]:
        (w1, bb1, gg1, bbe1), (w2, bb2, gg2, bbe2) = lc["mlp"]
        lfW, lfb = lc["final"]
        hb = _kp_block(feats, [(_pw(w1, D, D), _pp(bb1, gg1, bbe1, D))])
        bx = _kp_block(hb, [(_pw(w2, D, 64), _pp(bb2, gg2, bbe2, 64))],
                       final=(_pw(lfW, 64, 8), _pr(lfb, 8)))
        boxes.append(bx)

    logits = logits_p[:NKEY, :4]
    box_encodings = jnp.concatenate(
        [b[:NKEY, :7][:, None, :] for b in boxes], 1)
    return (logits, box_encodings)
